# Initial kernel scaffold; baseline (speedup 1.0000x reference)
import jax
import jax.numpy as jnp
from jax.experimental import pallas as pl


def kernel(data, float_table, out_scale):
    raise NotImplementedError("write your pallas kernel here")


# speedup vs baseline: 309.8932x; 309.8932x over previous
import functools

import jax
import jax.numpy as jnp
from jax import lax
from jax.experimental import pallas as pl
from jax.experimental.pallas import tpu as pltpu
from jax.experimental.pallas import tpu_sc as plsc

NC, NS, L = 2, 16, 16
NW = NC * NS

N = 16384 * 1024
PER_W = N // NW
CHUNK = 8192
NCHUNK = PER_W // CHUNK


def _sc_body(data_hbm, lut_hbm, out_hbm, lut_v, in_v, out_v):
    wid = lax.axis_index("s") * NC + lax.axis_index("c")
    base = wid * PER_W

    pltpu.sync_copy(lut_hbm, lut_v)

    @pl.loop(0, NCHUNK)
    def _chunk(g):
        off = base + g * CHUNK
        pltpu.sync_copy(data_hbm.at[pl.ds(off, CHUNK)], in_v)

        @pl.loop(0, CHUNK // L, unroll=4)
        def _vec(i):
            d = in_v[pl.ds(i * L, L)]
            out_v[pl.ds(i * L, L)] = plsc.load_gather(lut_v, [d + 128])

        pltpu.sync_copy(out_v, out_hbm.at[pl.ds(off, CHUNK)])


@functools.partial(
    pl.kernel,
    out_type=jax.ShapeDtypeStruct((N,), jnp.float32),
    mesh=plsc.VectorSubcoreMesh(
        core_axis_name="c", subcore_axis_name="s", num_cores=NC, num_subcores=NS
    ),
    scratch_types=[
        pltpu.VMEM((256,), jnp.float32),
        pltpu.VMEM((CHUNK,), jnp.int32),
        pltpu.VMEM((CHUNK,), jnp.float32),
    ],
    compiler_params=pltpu.CompilerParams(needs_layout_passes=False),
)
def _sc_lookup(data_hbm, lut_hbm, out_hbm, lut_v, in_v, out_v):
    _sc_body(data_hbm, lut_hbm, out_hbm, lut_v, in_v, out_v)


@jax.jit
def kernel(data, float_table, out_scale):
    table_int = jnp.round(float_table * 128.0).astype(jnp.int32)
    table_int = jnp.clip(table_int, -128, 127)
    lut = table_int.astype(jnp.float32) * out_scale[0]
    out = _sc_lookup(data.reshape(-1), lut)
    return out.reshape(data.shape)


# speedup vs baseline: 894.1819x; 2.8855x over previous
import functools

import jax
import jax.numpy as jnp
from jax import lax
from jax.experimental import pallas as pl
from jax.experimental.pallas import tpu as pltpu
from jax.experimental.pallas import tpu_sc as plsc

NC, NS, L = 2, 16, 16
NW = NC * NS

N = 16384 * 1024
PER_W = N // NW
CHUNK = 16384
NCHUNK = PER_W // CHUNK


def _sc_body(data_hbm, lut_hbm, out_hbm, lut_v, inb, outb, sin, sout):
    wid = lax.axis_index("s") * NC + lax.axis_index("c")
    base = wid * PER_W

    pltpu.sync_copy(lut_hbm, lut_v)

    for b in range(2):
        pltpu.async_copy(
            data_hbm.at[pl.ds(base + b * CHUNK, CHUNK)], inb[b], sin[b]
        )

    @pl.loop(0, NCHUNK, step=2)
    def _pair(g0):
        for b in range(2):
            g = g0 + b
            off = base + g * CHUNK

            pltpu.make_async_copy(
                data_hbm.at[pl.ds(off, CHUNK)], inb[b], sin[b]
            ).wait()

            @pl.when(g0 >= 2)
            def _():
                pltpu.make_async_copy(
                    outb[b], out_hbm.at[pl.ds(off - 2 * CHUNK, CHUNK)], sout[b]
                ).wait()

            @plsc.parallel_loop(0, CHUNK // L, unroll=8)
            def _vec(i):
                d = inb[b][pl.ds(i * L, L)]
                outb[b][pl.ds(i * L, L)] = plsc.load_gather(lut_v, [d + 128])

            pltpu.async_copy(outb[b], out_hbm.at[pl.ds(off, CHUNK)], sout[b])

            @pl.when(g0 + 2 < NCHUNK)
            def _():
                pltpu.async_copy(
                    data_hbm.at[pl.ds(off + 2 * CHUNK, CHUNK)], inb[b], sin[b]
                )

    for b in range(2):
        g = NCHUNK - 2 + b
        pltpu.make_async_copy(
            outb[b], out_hbm.at[pl.ds(base + g * CHUNK, CHUNK)], sout[b]
        ).wait()


@functools.partial(
    pl.kernel,
    out_type=jax.ShapeDtypeStruct((N,), jnp.float32),
    mesh=plsc.VectorSubcoreMesh(
        core_axis_name="c", subcore_axis_name="s", num_cores=NC, num_subcores=NS
    ),
    scratch_types=[
        pltpu.VMEM((256,), jnp.float32),
        pltpu.VMEM((CHUNK,), jnp.int32),
        pltpu.VMEM((CHUNK,), jnp.int32),
        pltpu.VMEM((CHUNK,), jnp.float32),
        pltpu.VMEM((CHUNK,), jnp.float32),
        pltpu.SemaphoreType.DMA,
        pltpu.SemaphoreType.DMA,
        pltpu.SemaphoreType.DMA,
        pltpu.SemaphoreType.DMA,
    ],
    compiler_params=pltpu.CompilerParams(needs_layout_passes=False),
)
def _sc_lookup(data_hbm, lut_hbm, out_hbm, lut_v, in0, in1, o0, o1, si0, si1, so0, so1):
    _sc_body(
        data_hbm, lut_hbm, out_hbm, lut_v,
        [in0, in1], [o0, o1], [si0, si1], [so0, so1],
    )


@jax.jit
def kernel(data, float_table, out_scale):
    table_int = jnp.round(float_table * 128.0).astype(jnp.int32)
    table_int = jnp.clip(table_int, -128, 127)
    lut = table_int.astype(jnp.float32) * out_scale[0]
    out = _sc_lookup(data.reshape(-1), lut)
    return out.reshape(data.shape)


# speedup vs baseline: 2295.8273x; 2.5675x over previous
import functools

import jax
import jax.numpy as jnp
from jax import lax
from jax.experimental import pallas as pl
from jax.experimental.pallas import tpu as pltpu
from jax.experimental.pallas import tpu_sc as plsc

NC, NS, L = 2, 16, 16
NW = NC * NS

ROWS, COLS = 16384, 1024
ROWS_W = ROWS // NW
CR = 16
NCHUNK = ROWS_W // CR
VEC_PER_CHUNK = CR * COLS // L


def _sc_body(data_hbm, lut_hbm, out_hbm, lut_v, inb, outb, sin, sout):
    wid = lax.axis_index("s") * NC + lax.axis_index("c")
    base = wid * ROWS_W

    pltpu.sync_copy(lut_hbm, lut_v)

    for b in range(2):
        pltpu.async_copy(
            data_hbm.at[pl.ds(base + b * CR, CR)], inb[b], sin[b]
        )

    @pl.loop(0, NCHUNK, step=2)
    def _pair(g0):
        for b in range(2):
            g = g0 + b
            row = base + g * CR

            pltpu.make_async_copy(
                data_hbm.at[pl.ds(row, CR)], inb[b], sin[b]
            ).wait()

            @pl.when(g0 >= 2)
            def _():
                pltpu.make_async_copy(
                    outb[b], out_hbm.at[pl.ds(row - 2 * CR, CR)], sout[b]
                ).wait()

            @plsc.parallel_loop(0, VEC_PER_CHUNK, unroll=8)
            def _vec(i):
                r = i >> 6
                c = (i & 63) << 4
                d = inb[b][r, pl.ds(c, L)]
                outb[b][r, pl.ds(c, L)] = plsc.load_gather(lut_v, [d + 128])

            pltpu.async_copy(outb[b], out_hbm.at[pl.ds(row, CR)], sout[b])

            @pl.when(g0 + 2 < NCHUNK)
            def _():
                pltpu.async_copy(
                    data_hbm.at[pl.ds(row + 2 * CR, CR)], inb[b], sin[b]
                )

    for b in range(2):
        row = base + (NCHUNK - 2 + b) * CR
        pltpu.make_async_copy(
            outb[b], out_hbm.at[pl.ds(row, CR)], sout[b]
        ).wait()


@functools.partial(
    pl.kernel,
    out_type=jax.ShapeDtypeStruct((ROWS, COLS), jnp.float32),
    mesh=plsc.VectorSubcoreMesh(
        core_axis_name="c", subcore_axis_name="s", num_cores=NC, num_subcores=NS
    ),
    scratch_types=[
        pltpu.VMEM((256,), jnp.float32),
        pltpu.VMEM((CR, COLS), jnp.int32),
        pltpu.VMEM((CR, COLS), jnp.int32),
        pltpu.VMEM((CR, COLS), jnp.float32),
        pltpu.VMEM((CR, COLS), jnp.float32),
        pltpu.SemaphoreType.DMA,
        pltpu.SemaphoreType.DMA,
        pltpu.SemaphoreType.DMA,
        pltpu.SemaphoreType.DMA,
    ],
    compiler_params=pltpu.CompilerParams(needs_layout_passes=False),
)
def _sc_lookup(data_hbm, lut_hbm, out_hbm, lut_v, in0, in1, o0, o1, si0, si1, so0, so1):
    _sc_body(
        data_hbm, lut_hbm, out_hbm, lut_v,
        [in0, in1], [o0, o1], [si0, si1], [so0, so1],
    )


@jax.jit
def kernel(data, float_table, out_scale):
    table_int = jnp.round(float_table * 128.0).astype(jnp.int32)
    table_int = jnp.clip(table_int, -128, 127)
    lut = table_int.astype(jnp.float32) * out_scale[0]
    return _sc_lookup(data, lut)


# speedup vs baseline: 2302.5299x; 1.0029x over previous
import functools

import jax
import jax.numpy as jnp
from jax import lax
from jax.experimental import pallas as pl
from jax.experimental.pallas import tpu as pltpu
from jax.experimental.pallas import tpu_sc as plsc

NC, NS, L = 2, 16, 16
NW = NC * NS

ROWS, COLS = 16384, 1024
ROWS_W = ROWS // NW
CR = 16
NCHUNK = ROWS_W // CR
VEC_PER_CHUNK = CR * COLS // L


def _sc_body(data_hbm, lut_hbm, out_hbm, lut_v, inb, outb, sin, sout):
    wid = lax.axis_index("s") * NC + lax.axis_index("c")
    base = wid * ROWS_W

    pltpu.sync_copy(lut_hbm, lut_v)

    for b in range(2):
        pltpu.async_copy(
            data_hbm.at[pl.ds(base + b * CR, CR)], inb[b], sin[b]
        )

    @pl.loop(0, NCHUNK, step=2)
    def _pair(g0):
        for b in range(2):
            g = g0 + b
            row = base + g * CR

            pltpu.make_async_copy(
                data_hbm.at[pl.ds(row, CR)], inb[b], sin[b]
            ).wait()

            @pl.when(g0 >= 2)
            def _():
                pltpu.make_async_copy(
                    outb[b], out_hbm.at[pl.ds(row - 2 * CR, CR)], sout[b]
                ).wait()

            @plsc.parallel_loop(0, VEC_PER_CHUNK, unroll=16)
            def _vec(i):
                r = i >> 6
                c = (i & 63) << 4
                d = inb[b][r, pl.ds(c, L)]
                outb[b][r, pl.ds(c, L)] = plsc.load_gather(lut_v, [d + 128])

            pltpu.async_copy(outb[b], out_hbm.at[pl.ds(row, CR)], sout[b])

            @pl.when(g0 + 2 < NCHUNK)
            def _():
                pltpu.async_copy(
                    data_hbm.at[pl.ds(row + 2 * CR, CR)], inb[b], sin[b]
                )

    for b in range(2):
        row = base + (NCHUNK - 2 + b) * CR
        pltpu.make_async_copy(
            outb[b], out_hbm.at[pl.ds(row, CR)], sout[b]
        ).wait()


@functools.partial(
    pl.kernel,
    out_type=jax.ShapeDtypeStruct((ROWS, COLS), jnp.float32),
    mesh=plsc.VectorSubcoreMesh(
        core_axis_name="c", subcore_axis_name="s", num_cores=NC, num_subcores=NS
    ),
    scratch_types=[
        pltpu.VMEM((256,), jnp.float32),
        pltpu.VMEM((CR, COLS), jnp.int32),
        pltpu.VMEM((CR, COLS), jnp.int32),
        pltpu.VMEM((CR, COLS), jnp.float32),
        pltpu.VMEM((CR, COLS), jnp.float32),
        pltpu.SemaphoreType.DMA,
        pltpu.SemaphoreType.DMA,
        pltpu.SemaphoreType.DMA,
        pltpu.SemaphoreType.DMA,
    ],
    compiler_params=pltpu.CompilerParams(needs_layout_passes=False),
)
def _sc_lookup(data_hbm, lut_hbm, out_hbm, lut_v, in0, in1, o0, o1, si0, si1, so0, so1):
    _sc_body(
        data_hbm, lut_hbm, out_hbm, lut_v,
        [in0, in1], [o0, o1], [si0, si1], [so0, so1],
    )


@jax.jit
def kernel(data, float_table, out_scale):
    table_int = jnp.round(float_table * 128.0).astype(jnp.int32)
    table_int = jnp.clip(table_int, -128, 127)
    lut = table_int.astype(jnp.float32) * out_scale[0]
    return _sc_lookup(data, lut)
